# Initial kernel scaffold; baseline (speedup 1.0000x reference)
import jax
import jax.numpy as jnp
from jax.experimental import pallas as pl


def kernel(input, cols, mask):
    raise NotImplementedError("write your pallas kernel here")


# speedup vs baseline: 839.1266x; 839.1266x over previous
import functools

import jax
import jax.numpy as jnp
from jax.experimental import pallas as pl
from jax.experimental.pallas import tpu as pltpu


def _pool_kernel(x_ref, idx_ref, o_ref, *, O, J):
    n_in = x_ref.shape[1]
    n_out = o_ref.shape[1]
    x = x_ref[0]

    row = jax.lax.broadcasted_iota(jnp.int32, (n_out, n_out), 0)
    col = jax.lax.broadcasted_iota(jnp.int32, (n_out, n_out), 1)
    d_ij = (row // J) == (col // J)
    d_i = (row // (O * J)) == (col // (O * J))
    d_j = ((row // J) % O) == ((col // J) % O)

    riota = jax.lax.broadcasted_iota(jnp.int32, (n_in, n_out), 0)

    def gsel(cls, oi, oj):
        a = x.reshape(O, 4 * O * J, n_in)[:, 2 * O * J * oi:2 * O * J * (oi + 1), :]
        b = a.reshape(O, O, 2 * J, n_in)[:, :, J * oj:J * oj + J, :]
        xr = b.reshape(n_out, n_in)
        sel = jnp.where(riota == idx_ref[cls], 1.0, 0.0)
        return jax.lax.dot(xr, sel, preferred_element_type=jnp.float32)

    g_a = gsel(0, 1, 1)
    g_b = gsel(1, 0, 0)
    g_c = gsel(2, 0, 1)
    g_d = gsel(3, 1, 0)
    o_ref[0] = (g_a + jnp.where(d_ij, g_b, 0.0) + jnp.where(d_i, g_c, 0.0)
                + jnp.where(d_j, g_d, 0.0))


def kernel(input, cols, mask):
    B, n_in, _ = input.shape
    K, n_out = cols.shape
    O = int(round(K ** 0.5)) - 1
    J = n_out // (O * O)

    safe = jnp.where(mask, cols, 0).astype(jnp.int32)
    idx_a = safe[0]
    idx_b = jnp.sum(safe[1:1 + O * O], axis=0)
    idx_c = jnp.sum(safe[1 + O * O:1 + O * O + O], axis=0)
    idx_d = jnp.sum(safe[1 + O * O + O:], axis=0)
    idx4 = jnp.stack([idx_a, idx_b, idx_c, idx_d]).reshape(4, 1, n_out)

    fn = pl.pallas_call(
        functools.partial(_pool_kernel, O=O, J=J),
        grid=(B,),
        in_specs=[
            pl.BlockSpec((1, n_in, n_in), lambda b: (b, 0, 0)),
            pl.BlockSpec((4, 1, n_out), lambda b: (0, 0, 0)),
        ],
        out_specs=pl.BlockSpec((1, n_out, n_out), lambda b: (b, 0, 0)),
        out_shape=jax.ShapeDtypeStruct((B, n_out, n_out), jnp.float32),
        compiler_params=pltpu.CompilerParams(
            dimension_semantics=("parallel",),
        ),
        name="pool2d_density3d",
    )
    return fn(input, idx4)
